# baseline (speedup 1.0000x reference)
import jax
import jax.numpy as jnp
from jax import lax
from jax.experimental import pallas as pl
from jax.experimental.pallas import tpu as pltpu
from jax.experimental.pallas import tpu_sc as plsc

NC = 2
NS = 16
NW = NC * NS
CHUNK = 128
DIM = 64
SCALE = 8.0

B, S = 4096, 200
TOTAL = B * S
NCHUNKS = TOTAL // CHUNK
CPW = NCHUNKS // NW


def _body(x_hbm, table_hbm, out_hbm, idx_v, rows_v, sem_g):
    wid = lax.axis_index("s") * NC + lax.axis_index("c")
    base = wid * CPW
    pltpu.sync_copy(x_hbm.at[pl.ds(base, CPW)], idx_v)

    def chunk_body(j, _):
        pltpu.async_copy(table_hbm.at[idx_v.at[j]], rows_v, sem_g).wait()

        def srow(r, _):
            for k in range(DIM // 16):
                sl = pl.ds(16 * k, 16)
                rows_v[r, sl] = rows_v[r, sl] * SCALE
            return 0

        lax.fori_loop(0, CHUNK, srow, 0)

        pltpu.sync_copy(rows_v, out_hbm.at[base + j])
        return 0

    lax.fori_loop(0, CPW, chunk_body, 0)


def kernel(x, table):
    x2d = x.astype(jnp.int32).reshape(NCHUNKS, CHUNK)
    mesh = plsc.VectorSubcoreMesh(core_axis_name="c", subcore_axis_name="s")
    out = pl.kernel(
        _body,
        mesh=mesh,
        out_type=jax.ShapeDtypeStruct((NCHUNKS, CHUNK, DIM), jnp.float32),
        scratch_types=[
            pltpu.VMEM((CPW, CHUNK), jnp.int32),
            pltpu.VMEM((CHUNK, DIM), jnp.float32),
            pltpu.SemaphoreType.DMA,
        ],
        compiler_params=pltpu.CompilerParams(use_tc_tiling_on_sc=False),
    )(x2d, table)
    return out.reshape(B, S, DIM)


# speedup vs baseline: 1.2126x; 1.2126x over previous
import jax
import jax.numpy as jnp
from jax import lax
from jax.experimental import pallas as pl
from jax.experimental.pallas import tpu as pltpu
from jax.experimental.pallas import tpu_sc as plsc

NC = 2
NS = 16
NW = NC * NS
CHUNK = 128
DIM = 64
SCALE = 8.0
NBUF = 4

B, S = 4096, 200
TOTAL = B * S
NCHUNKS = TOTAL // CHUNK
CPW = NCHUNKS // NW
NSTEP = CPW // NBUF


def _body(x_hbm, table_hbm, out_hbm, idx_v, gbuf, sbuf, sem_g, sem_s):
    wid = lax.axis_index("s") * NC + lax.axis_index("c")
    base = wid * CPW
    pltpu.sync_copy(x_hbm.at[pl.ds(base, CPW)], idx_v)

    def start_gather(j, b):
        pltpu.async_copy(table_hbm.at[idx_v.at[j]], gbuf.at[b], sem_g.at[b])

    def wait_gather(j, b):
        pltpu.make_async_copy(table_hbm.at[idx_v.at[j]], gbuf.at[b],
                              sem_g.at[b]).wait()

    def start_store(j, b):
        pltpu.async_copy(sbuf.at[b], out_hbm.at[base + j], sem_s.at[b])

    def wait_store(j, b):
        pltpu.make_async_copy(sbuf.at[b], out_hbm.at[base + j],
                              sem_s.at[b]).wait()

    def scale(b):
        def srow(r2, _):
            for dr in range(2):
                for k in range(DIM // 16):
                    sl = pl.ds(16 * k, 16)
                    sbuf[b, 2 * r2 + dr, sl] = gbuf[b, 2 * r2 + dr, sl] * SCALE
            return 0
        lax.fori_loop(0, CHUNK // 2, srow, 0)

    for b in range(NBUF):
        start_gather(b, b)

    for b in range(NBUF):
        wait_gather(b, b)
        scale(b)
        start_store(b, b)
        start_gather(b + NBUF, b)

    def step(i, _):
        for b in range(NBUF):
            j = i * NBUF + b
            wait_gather(j, b)
            wait_store(j - NBUF, b)
            scale(b)
            start_store(j, b)
            start_gather(j + NBUF, b)
        return 0

    lax.fori_loop(1, NSTEP - 1, step, 0)

    for b in range(NBUF):
        j = (NSTEP - 1) * NBUF + b
        wait_gather(j, b)
        wait_store(j - NBUF, b)
        scale(b)
        start_store(j, b)
    for b in range(NBUF):
        wait_store((NSTEP - 1) * NBUF + b, b)


def kernel(x, table):
    x2d = x.astype(jnp.int32).reshape(NCHUNKS, CHUNK)
    mesh = plsc.VectorSubcoreMesh(core_axis_name="c", subcore_axis_name="s")
    out = pl.kernel(
        _body,
        mesh=mesh,
        out_type=jax.ShapeDtypeStruct((NCHUNKS, CHUNK, DIM), jnp.float32),
        scratch_types=[
            pltpu.VMEM((CPW, CHUNK), jnp.int32),
            pltpu.VMEM((NBUF, CHUNK, DIM), jnp.float32),
            pltpu.VMEM((NBUF, CHUNK, DIM), jnp.float32),
            pltpu.SemaphoreType.DMA((NBUF,)),
            pltpu.SemaphoreType.DMA((NBUF,)),
        ],
        compiler_params=pltpu.CompilerParams(use_tc_tiling_on_sc=False),
    )(x2d, table)
    return out.reshape(B, S, DIM)
